# baseline (speedup 1.0000x reference)
import functools

import jax
import jax.numpy as jnp
from jax import lax
from jax.experimental import pallas as pl
from jax.experimental.pallas import tpu as pltpu
from jax.experimental.pallas import tpu_sc as plsc

B, L, S, D = 1024, 200, 50, 128
NR = 6

NC, NS = 2, 16
NW = NC * NS
CH = 128

def _sc_gather_body(tab_i, tab_u, idx_i_hbm, idx_u_hbm, out_i, out_u,
                    idx_vi, idx_vu, buf0, buf1, sem, w0, w1):
    wid = lax.axis_index("s") * NC + lax.axis_index("c")

    def run(tab, idx_hbm, idx_slab, out, chunks_w):
        base = wid * chunks_w * CH
        pltpu.sync_copy(idx_hbm.at[pl.ds(base, chunks_w * CH)], idx_slab)

        def idxr(j):
            return idx_slab.at[pl.ds(j * CH, CH)]

        def wait_w(buf, wsem):
            pltpu.make_async_copy(buf, out.at[pl.ds(base, CH)], wsem).wait()

        def pair(t, carry):
            a = 2 * t

            @pl.when(t > 0)
            def _():
                wait_w(buf0, w0)
                wait_w(buf1, w1)

            pltpu.async_copy(tab.at[idxr(a)], buf0, sem).wait()
            pltpu.async_copy(buf0, out.at[pl.ds(base + a * CH, CH)], w0)
            pltpu.async_copy(tab.at[idxr(a + 1)], buf1, sem).wait()
            pltpu.async_copy(buf1, out.at[pl.ds(base + (a + 1) * CH, CH)], w1)
            return carry

        lax.fori_loop(0, chunks_w // 2, pair, 0)
        if chunks_w % 2:
            a = chunks_w - 1
            if chunks_w > 1:
                wait_w(buf0, w0)
            pltpu.async_copy(tab.at[idxr(a)], buf0, sem).wait()
            pltpu.async_copy(buf0, out.at[pl.ds(base + a * CH, CH)], w0)
        wait_w(buf0, w0)
        if chunks_w > 1:
            wait_w(buf1, w1)

    run(tab_i, idx_i_hbm, idx_vi, out_i, idx_i_hbm.shape[0] // NW // CH)
    run(tab_u, idx_u_hbm, idx_vu, out_u, idx_u_hbm.shape[0] // NW // CH)


@jax.jit
def _sc_gather(tab_i, tab_u, idx_i, idx_u):
    mesh = plsc.VectorSubcoreMesh(core_axis_name="c", subcore_axis_name="s")
    width = tab_i.shape[1]
    return pl.kernel(
        _sc_gather_body,
        out_type=[
            jax.ShapeDtypeStruct((idx_i.shape[0], width), tab_i.dtype),
            jax.ShapeDtypeStruct((idx_u.shape[0], width), tab_u.dtype),
        ],
        mesh=mesh,
        scratch_types=[
            pltpu.VMEM((idx_i.shape[0] // NW,), jnp.int32),
            pltpu.VMEM((idx_u.shape[0] // NW,), jnp.int32),
            pltpu.VMEM((CH, width), tab_i.dtype),
            pltpu.VMEM((CH, width), tab_i.dtype),
            pltpu.SemaphoreType.DMA,
            pltpu.SemaphoreType.DMA,
            pltpu.SemaphoreType.DMA,
        ],
    )(tab_i, tab_u, idx_i, idx_u)


BLK = 64
RL = BLK * L
RS = BLK * S


def _tc_body(qa_ref, ur_ref, un_ref, pi_ref,
             emb_r8_ref, gvw1a_ref, gvw1b_ref, gvb1_ref, gvw2_ref, gvb2_ref,
             aIw1a_ref, aIw1b_ref, aIb1_ref, aIw2_ref, aIb2_ref, aIw3_ref, aIb3_ref,
             aSw1a_ref, aSw1b_ref, aSb1_ref, aSw2_ref, aSb2_ref, aSw3_ref, aSb3_ref,
             mw1a_ref, mw1b_ref, mb1_ref, mw2_ref, mb2_ref,
             segTI_ref, segTS_ref,
             out_ref):
    f32 = jnp.float32
    bf16 = jnp.bfloat16

    def dt(lhs, rhs, lc, rc):
        return lax.dot_general(lhs.astype(bf16), rhs.astype(bf16),
                               (((lc,), (rc,)), ((), ())),
                               preferred_element_type=f32)

    qa = qa_ref[...]
    piB = pi_ref[...]
    urT = ur_ref[...].reshape(1, RL)

    onehotT = (urT == lax.broadcasted_iota(jnp.int32, (8, RL), 0)) \
        .astype(bf16)
    tr8 = jnp.dot(emb_r8_ref[...], gvw1b_ref[...],
                  preferred_element_type=f32)
    hT = jnp.maximum(dt(gvw1a_ref[...], qa, 0, 1)
                     + dt(tr8, onehotT, 0, 0)
                     + gvb1_ref[...], 0.0).astype(bf16)
    xiaT = jnp.maximum(dt(gvw2_ref[...], hT, 0, 0)
                       + gvb2_ref[...], 0.0).astype(bf16)

    def attention(featT_for_mlp, lc_feat, value_dot, segT,
                  w1a_ref, w1b_ref, b1_ref, w2_ref, b2_ref, w3_ref, b3_ref):
        piWT = dt(w1b_ref[...], piB, 0, 1)
        aT = jnp.maximum(dt(w1a_ref[...], featT_for_mlp, 0, lc_feat)
                         + dt(piWT, segT, 1, 0)
                         + b1_ref[...], 0.0).astype(bf16)
        aT = jnp.maximum(dt(w2_ref[...], aT, 0, 0)
                         + b2_ref[...], 0.0).astype(bf16)
        logitT = dt(w3_ref[...], aT, 0, 0) + b3_ref[...]
        e = jnp.exp(logitT - jnp.max(logitT))
        AT = segT * e
        numer = value_dot(AT)
        den = lax.dot_general(AT, jnp.ones((AT.shape[1], 1), f32),
                              (((1,), (0,)), ((), ())),
                              preferred_element_type=f32)
        return numer / den

    segTI = segTI_ref[...]
    segTS = segTS_ref[...]
    hi_I = attention(xiaT, 0, lambda AT: dt(AT, xiaT, 1, 1), segTI,
                     aIw1a_ref, aIw1b_ref, aIb1_ref, aIw2_ref, aIb2_ref,
                     aIw3_ref, aIb3_ref)
    un = un_ref[...]
    hi_S = attention(un, 1, lambda AT: dt(AT, un, 1, 0), segTS,
                     aSw1a_ref, aSw1b_ref, aSb1_ref, aSw2_ref, aSb2_ref,
                     aSw3_ref, aSb3_ref)

    h2 = jnp.maximum(dt(hi_I, mw1a_ref[...], 1, 0)
                     + dt(hi_S, mw1b_ref[...], 1, 0)
                     + mb1_ref[...], 0.0)
    out_ref[...] = jnp.maximum(
        dt(h2, mw2_ref[...], 1, 0) + mb2_ref[...], 0.0)[:BLK]


def _seg_const(n_per_user, n_rows):
    return (jnp.arange(n_rows, dtype=jnp.int32)[None, :] // n_per_user
            == jnp.arange(BLK, dtype=jnp.int32)[:, None]).astype(jnp.float32)


def _tc_compute(qa, ur2, gu, nb, weights, interpret=False):
    n_blocks = nb // BLK
    pi_blk0 = nb * S // BLK
    row_spec = pl.BlockSpec((RL, D), lambda b: (b, 0))
    ur_spec = pl.BlockSpec((1, 1, RL), lambda b: (b, 0, 0))
    un_spec = pl.BlockSpec((RS, D), lambda b: (b, 0))
    pi_spec = pl.BlockSpec((BLK, D), lambda b: (b + pi_blk0, 0))

    def w_spec(w):
        return pl.BlockSpec(w.shape, lambda b: tuple(0 for _ in w.shape))

    consts = [_seg_const(L, RL), _seg_const(S, RS)]
    return pl.pallas_call(
        _tc_body,
        grid=(n_blocks,),
        in_specs=[row_spec, ur_spec, un_spec, pi_spec] +
                 [w_spec(w) for w in weights + consts],
        out_specs=pl.BlockSpec((BLK, D), lambda b: (b, 0)),
        out_shape=jax.ShapeDtypeStruct((nb, D), jnp.float32),
        compiler_params=pltpu.CompilerParams(
            dimension_semantics=("arbitrary",)),
        interpret=interpret,
    )(qa, ur2, gu, gu, *weights, *consts)


def _prep_weights(embed_r_w, gv_w1, gv_b1, gv_w2, gv_b2,
                  attI_w1, attI_b1, attI_w2, attI_b2, attI_w3, attI_b3,
                  attS_w1, attS_b1, attS_w2, attS_b2, attS_w3, attS_b3,
                  mlp_w1, mlp_b1, mlp_w2, mlp_b2):
    emb_r8 = jnp.zeros((8, D), jnp.float32).at[:NR].set(embed_r_w)
    row = lambda v: v.reshape(1, -1)
    col = lambda v: v.reshape(-1, 1)
    return [
        emb_r8, gv_w1[:D], gv_w1[D:], col(gv_b1), gv_w2, col(gv_b2),
        attI_w1[:D], attI_w1[D:], col(attI_b1), attI_w2, col(attI_b2),
        attI_w3, row(attI_b3),
        attS_w1[:D], attS_w1[D:], col(attS_b1), attS_w2, col(attS_b2),
        attS_w3, row(attS_b3),
        mlp_w1[:D], mlp_w1[D:], row(mlp_b1), mlp_w2, row(mlp_b2),
    ]


def kernel(nodes_u, history_u_lists_batch, social_adj_lists_batch,
           history_ur_lists_batch,
           embed_u_w, embed_i_w, embed_r_w,
           gv_w1, gv_b1, gv_w2, gv_b2,
           attI_w1, attI_b1, attI_w2, attI_b2, attI_w3, attI_b3,
           attS_w1, attS_b1, attS_w2, attS_b2, attS_w3, attS_b3,
           mlp_w1, mlp_b1, mlp_w2, mlp_b2):
    weights = _prep_weights(
        embed_r_w, gv_w1, gv_b1, gv_w2, gv_b2,
        attI_w1, attI_b1, attI_w2, attI_b2, attI_w3, attI_b3,
        attS_w1, attS_b1, attS_w2, attS_b2, attS_w3, attS_b3,
        mlp_w1, mlp_b1, mlp_w2, mlp_b2)

    phases = 4
    bp = B // phases
    grain = NW * CH
    i_pad = -(bp * L) % grain
    u_pad = -(bp * (S + 1)) % grain
    outs = []
    for p in range(phases):
        u0 = p * bp
        hist = lax.dynamic_slice_in_dim(history_u_lists_batch, u0, bp)
        soc = lax.dynamic_slice_in_dim(social_adj_lists_batch, u0, bp)
        nod = lax.dynamic_slice_in_dim(nodes_u, u0, bp)
        urp = lax.dynamic_slice_in_dim(history_ur_lists_batch, u0, bp)
        idx_i = jnp.concatenate([hist.reshape(bp * L),
                                 jnp.arange(i_pad, dtype=jnp.int32)])
        idx_u = jnp.concatenate([soc.reshape(bp * S), nod,
                                 jnp.arange(u_pad, dtype=jnp.int32)])
        qa, gu = _sc_gather(embed_i_w, embed_u_w, idx_i, idx_u)
        ur2 = urp.reshape(bp // BLK, 1, RL)
        outs.append(_tc_compute(qa, ur2, gu, bp, weights))
    return outs[0] if phases == 1 else jnp.concatenate(outs)


# speedup vs baseline: 1.0338x; 1.0338x over previous
import functools

import jax
import jax.numpy as jnp
from jax import lax
from jax.experimental import pallas as pl
from jax.experimental.pallas import tpu as pltpu
from jax.experimental.pallas import tpu_sc as plsc

B, L, S, D = 1024, 200, 50, 128
NR = 6

NC, NS = 2, 16
NW = NC * NS
CH = 128

def _sc_gather_body(tab_i, tab_u, idx_i_hbm, idx_u_hbm, out_i, out_u,
                    idx_vi, idx_vu, buf0, buf1, sem, w0, w1):
    wid = lax.axis_index("s") * NC + lax.axis_index("c")

    def run(tab, idx_hbm, idx_slab, out, chunks_w):
        base = wid * chunks_w * CH
        pltpu.sync_copy(idx_hbm.at[pl.ds(base, chunks_w * CH)], idx_slab)

        def idxr(j):
            return idx_slab.at[pl.ds(j * CH, CH)]

        def wait_w(buf, wsem):
            pltpu.make_async_copy(buf, out.at[pl.ds(base, CH)], wsem).wait()

        def pair(t, carry):
            a = 2 * t

            @pl.when(t > 0)
            def _():
                wait_w(buf0, w0)
                wait_w(buf1, w1)

            pltpu.async_copy(tab.at[idxr(a)], buf0, sem).wait()
            pltpu.async_copy(buf0, out.at[pl.ds(base + a * CH, CH)], w0)
            pltpu.async_copy(tab.at[idxr(a + 1)], buf1, sem).wait()
            pltpu.async_copy(buf1, out.at[pl.ds(base + (a + 1) * CH, CH)], w1)
            return carry

        lax.fori_loop(0, chunks_w // 2, pair, 0)
        if chunks_w % 2:
            a = chunks_w - 1
            if chunks_w > 1:
                wait_w(buf0, w0)
            pltpu.async_copy(tab.at[idxr(a)], buf0, sem).wait()
            pltpu.async_copy(buf0, out.at[pl.ds(base + a * CH, CH)], w0)
        wait_w(buf0, w0)
        if chunks_w > 1:
            wait_w(buf1, w1)

    run(tab_i, idx_i_hbm, idx_vi, out_i, idx_i_hbm.shape[0] // NW // CH)
    run(tab_u, idx_u_hbm, idx_vu, out_u, idx_u_hbm.shape[0] // NW // CH)


@jax.jit
def _sc_gather(tab_i, tab_u, idx_i, idx_u):
    mesh = plsc.VectorSubcoreMesh(core_axis_name="c", subcore_axis_name="s")
    width = tab_i.shape[1]
    return pl.kernel(
        _sc_gather_body,
        out_type=[
            jax.ShapeDtypeStruct((idx_i.shape[0], width), tab_i.dtype),
            jax.ShapeDtypeStruct((idx_u.shape[0], width), tab_u.dtype),
        ],
        mesh=mesh,
        scratch_types=[
            pltpu.VMEM((idx_i.shape[0] // NW,), jnp.int32),
            pltpu.VMEM((idx_u.shape[0] // NW,), jnp.int32),
            pltpu.VMEM((CH, width), tab_i.dtype),
            pltpu.VMEM((CH, width), tab_i.dtype),
            pltpu.SemaphoreType.DMA,
            pltpu.SemaphoreType.DMA,
            pltpu.SemaphoreType.DMA,
        ],
    )(tab_i, tab_u, idx_i, idx_u)


BLK = 64
RL = BLK * L
RS = BLK * S


def _tc_body(qa_ref, ur_ref, un_ref, pi_ref,
             emb_r8_ref, gvw1a_ref, gvw1b_ref, gvb1_ref, gvw2_ref, gvb2_ref,
             aIw1a_ref, aIw1b_ref, aIb1_ref, aIw2_ref, aIb2_ref, aIw3_ref, aIb3_ref,
             aSw1a_ref, aSw1b_ref, aSb1_ref, aSw2_ref, aSb2_ref, aSw3_ref, aSb3_ref,
             mw1a_ref, mw1b_ref, mb1_ref, mw2_ref, mb2_ref,
             out_ref):
    f32 = jnp.float32
    bf16 = jnp.bfloat16

    def dt(lhs, rhs, lc, rc):
        return lax.dot_general(lhs.astype(bf16), rhs.astype(bf16),
                               (((lc,), (rc,)), ((), ())),
                               preferred_element_type=f32)

    qa = qa_ref[...]
    piB = pi_ref[...]
    urT = ur_ref[...].reshape(1, RL)

    onehotT = (urT == lax.broadcasted_iota(jnp.int32, (8, RL), 0)) \
        .astype(bf16)
    tr8 = jnp.dot(emb_r8_ref[...], gvw1b_ref[...],
                  preferred_element_type=f32)
    hT = jnp.maximum(dt(gvw1a_ref[...], qa, 0, 1)
                     + dt(tr8, onehotT, 0, 0)
                     + gvb1_ref[...], 0.0).astype(bf16)
    xiaT = jnp.maximum(dt(gvw2_ref[...], hT, 0, 0)
                       + gvb2_ref[...], 0.0).astype(bf16)

    def attention(featT_for_mlp, lc_feat, value_dot, segT,
                  w1a_ref, w1b_ref, b1_ref, w2_ref, b2_ref, w3_ref, b3_ref):
        piWT = dt(w1b_ref[...], piB, 0, 1)
        aT = jnp.maximum(dt(w1a_ref[...], featT_for_mlp, 0, lc_feat)
                         + dt(piWT, segT, 1, 0)
                         + b1_ref[...], 0.0).astype(bf16)
        aT = jnp.maximum(dt(w2_ref[...], aT, 0, 0)
                         + b2_ref[...], 0.0).astype(bf16)
        logitT = dt(w3_ref[...], aT, 0, 0) + b3_ref[...]
        e = jnp.exp(logitT - jnp.max(logitT))
        AT = segT * e
        numer = value_dot(AT)
        den = jnp.sum(AT, axis=1, keepdims=True)
        return numer / den

    segTI = _seg_iota(L, RL)
    segTS = _seg_iota(S, RS)
    hi_I = attention(xiaT, 0, lambda AT: dt(AT, xiaT, 1, 1), segTI,
                     aIw1a_ref, aIw1b_ref, aIb1_ref, aIw2_ref, aIb2_ref,
                     aIw3_ref, aIb3_ref)
    un = un_ref[...]
    hi_S = attention(un, 1, lambda AT: dt(AT, un, 1, 0), segTS,
                     aSw1a_ref, aSw1b_ref, aSb1_ref, aSw2_ref, aSb2_ref,
                     aSw3_ref, aSb3_ref)

    h2 = jnp.maximum(dt(hi_I, mw1a_ref[...], 1, 0)
                     + dt(hi_S, mw1b_ref[...], 1, 0)
                     + mb1_ref[...], 0.0)
    out_ref[...] = jnp.maximum(
        dt(h2, mw2_ref[...], 1, 0) + mb2_ref[...], 0.0)[:BLK]


def _seg_iota(n_per_user, n_rows):
    return (lax.broadcasted_iota(jnp.int32, (BLK, n_rows), 1) // n_per_user
            == lax.broadcasted_iota(jnp.int32, (BLK, n_rows), 0)
            ).astype(jnp.float32)


def _tc_compute(qa, ur2, gu, nb, weights, interpret=False):
    n_blocks = nb // BLK
    pi_blk0 = nb * S // BLK
    row_spec = pl.BlockSpec((RL, D), lambda b: (b, 0))
    ur_spec = pl.BlockSpec((1, 1, RL), lambda b: (b, 0, 0))
    un_spec = pl.BlockSpec((RS, D), lambda b: (b, 0))
    pi_spec = pl.BlockSpec((BLK, D), lambda b: (b + pi_blk0, 0))

    def w_spec(w):
        return pl.BlockSpec(w.shape, lambda b: tuple(0 for _ in w.shape))

    return pl.pallas_call(
        _tc_body,
        grid=(n_blocks,),
        in_specs=[row_spec, ur_spec, un_spec, pi_spec] +
                 [w_spec(w) for w in weights],
        out_specs=pl.BlockSpec((BLK, D), lambda b: (b, 0)),
        out_shape=jax.ShapeDtypeStruct((nb, D), jnp.float32),
        compiler_params=pltpu.CompilerParams(
            dimension_semantics=("arbitrary",)),
        interpret=interpret,
    )(qa, ur2, gu, gu, *weights)


def _prep_weights(embed_r_w, gv_w1, gv_b1, gv_w2, gv_b2,
                  attI_w1, attI_b1, attI_w2, attI_b2, attI_w3, attI_b3,
                  attS_w1, attS_b1, attS_w2, attS_b2, attS_w3, attS_b3,
                  mlp_w1, mlp_b1, mlp_w2, mlp_b2):
    emb_r8 = jnp.zeros((8, D), jnp.float32).at[:NR].set(embed_r_w)
    row = lambda v: v.reshape(1, -1)
    col = lambda v: v.reshape(-1, 1)
    return [
        emb_r8, gv_w1[:D], gv_w1[D:], col(gv_b1), gv_w2, col(gv_b2),
        attI_w1[:D], attI_w1[D:], col(attI_b1), attI_w2, col(attI_b2),
        attI_w3, row(attI_b3),
        attS_w1[:D], attS_w1[D:], col(attS_b1), attS_w2, col(attS_b2),
        attS_w3, row(attS_b3),
        mlp_w1[:D], mlp_w1[D:], row(mlp_b1), mlp_w2, row(mlp_b2),
    ]


def kernel(nodes_u, history_u_lists_batch, social_adj_lists_batch,
           history_ur_lists_batch,
           embed_u_w, embed_i_w, embed_r_w,
           gv_w1, gv_b1, gv_w2, gv_b2,
           attI_w1, attI_b1, attI_w2, attI_b2, attI_w3, attI_b3,
           attS_w1, attS_b1, attS_w2, attS_b2, attS_w3, attS_b3,
           mlp_w1, mlp_b1, mlp_w2, mlp_b2):
    weights = _prep_weights(
        embed_r_w, gv_w1, gv_b1, gv_w2, gv_b2,
        attI_w1, attI_b1, attI_w2, attI_b2, attI_w3, attI_b3,
        attS_w1, attS_b1, attS_w2, attS_b2, attS_w3, attS_b3,
        mlp_w1, mlp_b1, mlp_w2, mlp_b2)

    phases = 4
    bp = B // phases
    grain = NW * CH
    i_pad = -(bp * L) % grain
    u_pad = -(bp * (S + 1)) % grain
    outs = []
    for p in range(phases):
        u0 = p * bp
        hist = lax.dynamic_slice_in_dim(history_u_lists_batch, u0, bp)
        soc = lax.dynamic_slice_in_dim(social_adj_lists_batch, u0, bp)
        nod = lax.dynamic_slice_in_dim(nodes_u, u0, bp)
        urp = lax.dynamic_slice_in_dim(history_ur_lists_batch, u0, bp)
        idx_i = jnp.concatenate([hist.reshape(bp * L),
                                 jnp.arange(i_pad, dtype=jnp.int32)])
        idx_u = jnp.concatenate([soc.reshape(bp * S), nod,
                                 jnp.arange(u_pad, dtype=jnp.int32)])
        qa, gu = _sc_gather(embed_i_w, embed_u_w, idx_i, idx_u)
        ur2 = urp.reshape(bp // BLK, 1, RL)
        outs.append(_tc_compute(qa, ur2, gu, bp, weights))
    return outs[0] if phases == 1 else jnp.concatenate(outs)
